# initial kernel scaffold (unmeasured)
import functools

import jax
import jax.numpy as jnp
from jax import lax
from jax.experimental import pallas as pl
from jax.experimental.pallas import tpu as pltpu

N_DEV = 8
M = 2048
K_IN = 2048
N_OUT = 2048
K_CHUNK = 512

CHUNK_ROWS = M // N_DEV

RS_STEPS = [
    (4, 2, 1024, 0),
    (2, 1, 512, 1024),
    (1, 0, 256, 1536),
]
AG_STEPS = [
    (1, 0, 256, 0),
    (2, 1, 512, 256),
    (4, 2, 1024, 768),
]
STAGE_ROWS = 1792


def _label(v):
    lo = v & 3
    return (v & 4) | (lo ^ (lo >> 1))


def _cast_body(x_ref, o_ref):
    o_ref[...] = x_ref[...].astype(jnp.bfloat16)


def _mlp_body(xbf_ref, w1_ref, w2_ref, out_ref):
    h = jnp.dot(
        xbf_ref[...],
        w1_ref[...].astype(jnp.bfloat16),
        preferred_element_type=jnp.float32,
    )
    h = jnp.maximum(h, 0.0).astype(jnp.bfloat16)
    contrib = jnp.dot(
        h,
        w2_ref[...].astype(jnp.bfloat16),
        preferred_element_type=jnp.float32,
    )
    k = pl.program_id(0)

    @pl.when(k == 0)
    def _():
        out_ref[...] = contrib

    @pl.when(k > 0)
    def _():
        out_ref[...] += contrib


def _allreduce_body(p_ref, out_ref, send_ref, rs_ref, ag_ref,
                    send_sems, recv_sems):
    i = lax.axis_index("i")
    l = _label(i)

    out_ref[...] = p_ref[...]

    start = l * 0
    for s, (mask, bitpos, rows, off) in enumerate(RS_STEPS):
        bit = (l >> bitpos) & 1
        keep_start = start + bit * rows
        send_start = start + (1 - bit) * rows
        partner = _label(l ^ mask)

        send_ref[pl.ds(0, rows), :] = (
            out_ref[pl.ds(send_start, rows), :].astype(jnp.bfloat16)
        )
        rdma = pltpu.make_async_remote_copy(
            src_ref=send_ref.at[pl.ds(0, rows), :],
            dst_ref=rs_ref.at[pl.ds(off, rows), :],
            send_sem=send_sems.at[s],
            recv_sem=recv_sems.at[s],
            device_id=(partner,),
            device_id_type=pl.DeviceIdType.MESH,
        )
        rdma.start()
        rdma.wait()

        out_ref[pl.ds(keep_start, rows), :] += (
            rs_ref[pl.ds(off, rows), :].astype(jnp.float32)
        )
        start = keep_start

    cur_start = start
    for s, (mask, bitpos, rows, off) in enumerate(AG_STEPS):
        bit = (l >> bitpos) & 1
        partner = _label(l ^ mask)

        send_ref[pl.ds(0, rows), :] = (
            out_ref[pl.ds(cur_start, rows), :].astype(jnp.bfloat16)
        )
        rdma = pltpu.make_async_remote_copy(
            src_ref=send_ref.at[pl.ds(0, rows), :],
            dst_ref=ag_ref.at[pl.ds(off, rows), :],
            send_sem=send_sems.at[3 + s],
            recv_sem=recv_sems.at[3 + s],
            device_id=(partner,),
            device_id_type=pl.DeviceIdType.MESH,
        )
        rdma.start()
        rdma.wait()

        new_start = cur_start - bit * rows
        partner_start = new_start + (1 - bit) * rows
        out_ref[pl.ds(partner_start, rows), :] = (
            ag_ref[pl.ds(off, rows), :].astype(jnp.float32)
        )
        cur_start = new_start


def kernel(x, W1, W2):
    hidden = W1.shape[1]
    n_k = hidden // K_CHUNK

    xbf = pl.pallas_call(
        _cast_body,
        out_shape=jax.ShapeDtypeStruct((M, K_IN), jnp.bfloat16),
        in_specs=[pl.BlockSpec(memory_space=pltpu.VMEM)],
        out_specs=pl.BlockSpec(memory_space=pltpu.VMEM),
    )(x)

    partial = pl.pallas_call(
        _mlp_body,
        grid=(n_k,),
        in_specs=[
            pl.BlockSpec((M, K_IN), lambda k: (0, 0)),
            pl.BlockSpec((K_IN, K_CHUNK), lambda k: (0, k)),
            pl.BlockSpec((K_CHUNK, N_OUT), lambda k: (k, 0)),
        ],
        out_specs=pl.BlockSpec((M, N_OUT), lambda k: (0, 0)),
        out_shape=jax.ShapeDtypeStruct((M, N_OUT), jnp.float32),
    )(xbf, W1, W2)

    out = pl.pallas_call(
        _allreduce_body,
        out_shape=jax.ShapeDtypeStruct((M, N_OUT), jnp.float32),
        in_specs=[pl.BlockSpec(memory_space=pltpu.VMEM)],
        out_specs=pl.BlockSpec(memory_space=pltpu.VMEM),
        scratch_shapes=[
            pltpu.VMEM((1024, N_OUT), jnp.bfloat16),
            pltpu.VMEM((STAGE_ROWS, N_OUT), jnp.bfloat16),
            pltpu.VMEM((STAGE_ROWS, N_OUT), jnp.bfloat16),
            pltpu.SemaphoreType.DMA((6,)),
            pltpu.SemaphoreType.DMA((6,)),
        ],
        compiler_params=pltpu.CompilerParams(collective_id=0),
    )(partial)
    return out


# baseline (device time: 308793 ns/iter reference)
import functools

import jax
import jax.numpy as jnp
from jax import lax
from jax.experimental import pallas as pl
from jax.experimental.pallas import tpu as pltpu

N_DEV = 8
M = 2048
K_IN = 2048
N_OUT = 2048
K_CHUNK = 512

CHUNK_ROWS = M // N_DEV

RS_STEPS = [
    (4, 2, 1024, 0),
    (2, 1, 512, 1024),
    (1, 0, 256, 1536),
]
AG_STEPS = [
    (1, 0, 256, 0),
    (2, 1, 512, 256),
    (4, 2, 1024, 768),
]
STAGE_ROWS = 1792


def _label(v):
    lo = v & 3
    return (v & 4) | (lo ^ (lo >> 1))


def _cast_body(x_ref, o_ref):
    o_ref[...] = x_ref[...].astype(jnp.bfloat16)


def _mlp_body(xbf_ref, w1_ref, w2_ref, out_ref):
    h = jnp.dot(
        xbf_ref[...],
        w1_ref[...].astype(jnp.bfloat16),
        preferred_element_type=jnp.float32,
    )
    h = jnp.maximum(h, 0.0).astype(jnp.bfloat16)
    contrib = jnp.dot(
        h,
        w2_ref[...].astype(jnp.bfloat16),
        preferred_element_type=jnp.float32,
    )
    k = pl.program_id(0)

    @pl.when(k == 0)
    def _():
        out_ref[...] = contrib

    @pl.when(k > 0)
    def _():
        out_ref[...] += contrib


def _allreduce_body(p_ref, out_ref, send_ref, rs_ref, ag_ref,
                    send_sems, recv_sems):
    i = lax.axis_index("i")
    l = _label(i)

    barrier_sem = pltpu.get_barrier_semaphore()
    for mask in (1, 2, 4):
        pl.semaphore_signal(
            barrier_sem, inc=1,
            device_id=(_label(l ^ mask),),
            device_id_type=pl.DeviceIdType.MESH,
        )
    pl.semaphore_wait(barrier_sem, 3)

    out_ref[...] = p_ref[...]

    start = l * 0
    for s, (mask, bitpos, rows, off) in enumerate(RS_STEPS):
        bit = (l >> bitpos) & 1
        keep_start = start + bit * rows
        send_start = start + (1 - bit) * rows
        partner = _label(l ^ mask)

        send_ref[pl.ds(0, rows), :] = (
            out_ref[pl.ds(send_start, rows), :].astype(jnp.bfloat16)
        )
        rdma = pltpu.make_async_remote_copy(
            src_ref=send_ref.at[pl.ds(0, rows), :],
            dst_ref=rs_ref.at[pl.ds(off, rows), :],
            send_sem=send_sems.at[s],
            recv_sem=recv_sems.at[s],
            device_id=(partner,),
            device_id_type=pl.DeviceIdType.MESH,
        )
        rdma.start()
        rdma.wait()

        out_ref[pl.ds(keep_start, rows), :] += (
            rs_ref[pl.ds(off, rows), :].astype(jnp.float32)
        )
        start = keep_start

    cur_start = start
    for s, (mask, bitpos, rows, off) in enumerate(AG_STEPS):
        bit = (l >> bitpos) & 1
        partner = _label(l ^ mask)

        send_ref[pl.ds(0, rows), :] = (
            out_ref[pl.ds(cur_start, rows), :].astype(jnp.bfloat16)
        )
        rdma = pltpu.make_async_remote_copy(
            src_ref=send_ref.at[pl.ds(0, rows), :],
            dst_ref=ag_ref.at[pl.ds(off, rows), :],
            send_sem=send_sems.at[3 + s],
            recv_sem=recv_sems.at[3 + s],
            device_id=(partner,),
            device_id_type=pl.DeviceIdType.MESH,
        )
        rdma.start()
        rdma.wait()

        new_start = cur_start - bit * rows
        partner_start = new_start + (1 - bit) * rows
        out_ref[pl.ds(partner_start, rows), :] = (
            ag_ref[pl.ds(off, rows), :].astype(jnp.float32)
        )
        cur_start = new_start


def kernel(x, W1, W2):
    hidden = W1.shape[1]
    n_k = hidden // K_CHUNK

    xbf = pl.pallas_call(
        _cast_body,
        out_shape=jax.ShapeDtypeStruct((M, K_IN), jnp.bfloat16),
        in_specs=[pl.BlockSpec(memory_space=pltpu.VMEM)],
        out_specs=pl.BlockSpec(memory_space=pltpu.VMEM),
    )(x)

    partial = pl.pallas_call(
        _mlp_body,
        grid=(n_k,),
        in_specs=[
            pl.BlockSpec((M, K_IN), lambda k: (0, 0)),
            pl.BlockSpec((K_IN, K_CHUNK), lambda k: (0, k)),
            pl.BlockSpec((K_CHUNK, N_OUT), lambda k: (k, 0)),
        ],
        out_specs=pl.BlockSpec((M, N_OUT), lambda k: (0, 0)),
        out_shape=jax.ShapeDtypeStruct((M, N_OUT), jnp.float32),
        compiler_params=pltpu.CompilerParams(
            vmem_limit_bytes=100 * 1024 * 1024
        ),
    )(xbf, W1, W2)

    out = pl.pallas_call(
        _allreduce_body,
        out_shape=jax.ShapeDtypeStruct((M, N_OUT), jnp.float32),
        in_specs=[pl.BlockSpec(memory_space=pltpu.VMEM)],
        out_specs=pl.BlockSpec(memory_space=pltpu.VMEM),
        scratch_shapes=[
            pltpu.VMEM((1024, N_OUT), jnp.bfloat16),
            pltpu.VMEM((STAGE_ROWS, N_OUT), jnp.bfloat16),
            pltpu.VMEM((STAGE_ROWS, N_OUT), jnp.bfloat16),
            pltpu.SemaphoreType.DMA((6,)),
            pltpu.SemaphoreType.DMA((6,)),
        ],
        compiler_params=pltpu.CompilerParams(
            vmem_limit_bytes=100 * 1024 * 1024,
            collective_id=0,
        ),
    )(partial)
    return out


# device time: 209970 ns/iter; 1.4707x vs baseline; 1.4707x over previous
import jax
import jax.numpy as jnp
from jax import lax
from jax.experimental import pallas as pl
from jax.experimental.pallas import tpu as pltpu

N_DEV = 8
M = 2048
K_IN = 2048
N_OUT = 2048
K_CHUNK = 512

SUBS = [
    dict(row0=0, rows=768, rs_bits=(2, 1, 0)),
    dict(row0=768, rows=640, rs_bits=(1, 0, 2)),
    dict(row0=1408, rows=640, rs_bits=(0, 2, 1)),
]
for _s in SUBS:
    _s["rs_sizes"] = [_s["rows"] >> (k + 1) for k in range(3)]
    _s["rs_offs"] = [0, _s["rs_sizes"][0], _s["rs_sizes"][0] + _s["rs_sizes"][1]]
    _s["ag_bits"] = tuple(reversed(_s["rs_bits"]))
    _s["ag_sizes"] = [_s["rows"] >> (3 - k) for k in range(3)]
    _s["ag_offs"] = [0, _s["ag_sizes"][0], _s["ag_sizes"][0] + _s["ag_sizes"][1]]
    _s["stage_rows"] = sum(_s["rs_sizes"])


def _label(v):
    lo = v & 3
    return (v & 4) | (lo ^ (lo >> 1))


def _cast_body(x_ref, o_ref):
    o_ref[...] = x_ref[...].astype(jnp.bfloat16)


def _mlp_body(xbf_ref, w1_ref, w2_ref, out_ref):
    h = jnp.dot(
        xbf_ref[...],
        w1_ref[...].astype(jnp.bfloat16),
        preferred_element_type=jnp.float32,
    )
    h = jnp.maximum(h, 0.0).astype(jnp.bfloat16)
    contrib = jnp.dot(
        h,
        w2_ref[...].astype(jnp.bfloat16),
        preferred_element_type=jnp.float32,
    )
    k = pl.program_id(0)

    @pl.when(k == 0)
    def _():
        out_ref[...] = contrib

    @pl.when(k > 0)
    def _():
        out_ref[...] += contrib


def _allreduce_body(p_ref, out_ref, *scratch):
    send_bufs = scratch[0:3]
    rs_stages = scratch[3:6]
    ag_stages = scratch[6:9]
    send_sems, recv_sems = scratch[9], scratch[10]

    i = lax.axis_index("i")
    l = _label(i)

    barrier_sem = pltpu.get_barrier_semaphore()
    for mask in (1, 2, 4):
        pl.semaphore_signal(
            barrier_sem, inc=1,
            device_id=(_label(l ^ mask),),
            device_id_type=pl.DeviceIdType.MESH,
        )
    pl.semaphore_wait(barrier_sem, 3)

    out_ref[...] = p_ref[...]

    starts = [l * 0 + sub["row0"] for sub in SUBS]
    for s in range(3):
        pending = []
        for j, sub in enumerate(SUBS):
            bitpos = sub["rs_bits"][s]
            rows = sub["rs_sizes"][s]
            off = sub["rs_offs"][s]
            bit = (l >> bitpos) & 1
            keep_start = starts[j] + bit * rows
            send_start = starts[j] + (1 - bit) * rows
            partner = _label(l ^ (1 << bitpos))

            send_bufs[j][pl.ds(0, rows), :] = (
                out_ref[pl.ds(send_start, rows), :].astype(jnp.bfloat16)
            )
            rdma = pltpu.make_async_remote_copy(
                src_ref=send_bufs[j].at[pl.ds(0, rows), :],
                dst_ref=rs_stages[j].at[pl.ds(off, rows), :],
                send_sem=send_sems.at[j * 6 + s],
                recv_sem=recv_sems.at[j * 6 + s],
                device_id=(partner,),
                device_id_type=pl.DeviceIdType.MESH,
            )
            rdma.start()
            pending.append((rdma, j, keep_start, rows, off))
            starts[j] = keep_start
        for rdma, j, keep_start, rows, off in pending:
            rdma.wait()
            out_ref[pl.ds(keep_start, rows), :] += (
                rs_stages[j][pl.ds(off, rows), :].astype(jnp.float32)
            )

    for s in range(3):
        pending = []
        for j, sub in enumerate(SUBS):
            bitpos = sub["ag_bits"][s]
            rows = sub["ag_sizes"][s]
            off = sub["ag_offs"][s]
            bit = (l >> bitpos) & 1
            partner = _label(l ^ (1 << bitpos))

            send_bufs[j][pl.ds(0, rows), :] = (
                out_ref[pl.ds(starts[j], rows), :].astype(jnp.bfloat16)
            )
            rdma = pltpu.make_async_remote_copy(
                src_ref=send_bufs[j].at[pl.ds(0, rows), :],
                dst_ref=ag_stages[j].at[pl.ds(off, rows), :],
                send_sem=send_sems.at[j * 6 + 3 + s],
                recv_sem=recv_sems.at[j * 6 + 3 + s],
                device_id=(partner,),
                device_id_type=pl.DeviceIdType.MESH,
            )
            rdma.start()
            new_start = starts[j] - bit * rows
            partner_start = new_start + (1 - bit) * rows
            pending.append((rdma, j, partner_start, rows, off))
            starts[j] = new_start
        for rdma, j, partner_start, rows, off in pending:
            rdma.wait()
            out_ref[pl.ds(partner_start, rows), :] = (
                ag_stages[j][pl.ds(off, rows), :].astype(jnp.float32)
            )


def kernel(x, W1, W2):
    hidden = W1.shape[1]
    n_k = hidden // K_CHUNK

    xbf = pl.pallas_call(
        _cast_body,
        out_shape=jax.ShapeDtypeStruct((M, K_IN), jnp.bfloat16),
        in_specs=[pl.BlockSpec(memory_space=pltpu.VMEM)],
        out_specs=pl.BlockSpec(memory_space=pltpu.VMEM),
    )(x)

    partial = pl.pallas_call(
        _mlp_body,
        grid=(n_k,),
        in_specs=[
            pl.BlockSpec((M, K_IN), lambda k: (0, 0)),
            pl.BlockSpec((K_IN, K_CHUNK), lambda k: (0, k)),
            pl.BlockSpec((K_CHUNK, N_OUT), lambda k: (k, 0)),
        ],
        out_specs=pl.BlockSpec((M, N_OUT), lambda k: (0, 0)),
        out_shape=jax.ShapeDtypeStruct((M, N_OUT), jnp.float32),
        compiler_params=pltpu.CompilerParams(
            vmem_limit_bytes=100 * 1024 * 1024
        ),
    )(xbf, W1, W2)

    scratch_shapes = (
        [pltpu.VMEM((sub["rs_sizes"][0], N_OUT), jnp.bfloat16) for sub in SUBS]
        + [pltpu.VMEM((sub["stage_rows"], N_OUT), jnp.bfloat16) for sub in SUBS]
        + [pltpu.VMEM((sub["stage_rows"], N_OUT), jnp.bfloat16) for sub in SUBS]
        + [pltpu.SemaphoreType.DMA((18,)), pltpu.SemaphoreType.DMA((18,))]
    )
    out = pl.pallas_call(
        _allreduce_body,
        out_shape=jax.ShapeDtypeStruct((M, N_OUT), jnp.float32),
        in_specs=[pl.BlockSpec(memory_space=pltpu.VMEM)],
        out_specs=pl.BlockSpec(memory_space=pltpu.VMEM),
        scratch_shapes=scratch_shapes,
        compiler_params=pltpu.CompilerParams(
            vmem_limit_bytes=100 * 1024 * 1024,
            collective_id=0,
        ),
    )(partial)
    return out
